# baseline (speedup 1.0000x reference)
import functools

import jax
import jax.numpy as jnp
from jax import lax
from jax.experimental import pallas as pl
from jax.experimental.pallas import tpu as pltpu
from jax.experimental.pallas import tpu_sc as plsc

N = 10000
E = 320000
D = 128
EPS = 1e-5

NC = 2
NS = 16
NW = NC * NS
EPT = E // NW
B = 125
NCH = EPT // B
G = 16
NP = 10240
RPT = NP // NS
RB = 10000

_MESH = plsc.VectorSubcoreMesh(
    core_axis_name="c", subcore_axis_name="s", num_cores=NC, num_subcores=NS
)


def _deg_call(dst):
    @functools.partial(
        pl.kernel,
        out_type=jax.ShapeDtypeStruct((NC, NP, D), jnp.float32),
        mesh=_MESH,
        scratch_types=[
            pltpu.VMEM((NCH, B), jnp.int32),
            pltpu.VMEM((B, D), jnp.float32),
            pltpu.VMEM_SHARED((NP, D), jnp.float32),
            pltpu.SemaphoreType.DMA,
        ],
    )
    def k(dst_hbm, out_hbm, dst_v, buf_v, deg_sh, isem):
        cid = lax.axis_index("c")
        sid = lax.axis_index("s")
        wid = cid * NS + sid

        pltpu.async_copy(dst_hbm.at[wid], dst_v, isem)

        def fillz(r, _):
            for j in range(D // 16):
                buf_v[r, pl.ds(j * 16, 16)] = jnp.zeros((16,), jnp.float32)
            return 0

        lax.fori_loop(0, B, fillz, 0)
        for kblk in range(RPT // B):
            pltpu.sync_copy(buf_v, deg_sh.at[pl.ds(sid * RPT + kblk * B, B)])
        if RPT % B:
            pltpu.sync_copy(
                buf_v.at[pl.ds(0, RPT % B)],
                deg_sh.at[pl.ds(sid * RPT + (RPT // B) * B, RPT % B)],
            )

        def fillo(r, _):
            for j in range(D // 16):
                buf_v[r, pl.ds(j * 16, 16)] = jnp.ones((16,), jnp.float32)
            return 0

        lax.fori_loop(0, B, fillo, 0)
        pltpu.make_async_copy(dst_hbm.at[wid], dst_v, isem).wait()
        plsc.subcore_barrier()

        def chunk(c, _):
            pltpu.sync_copy(buf_v, deg_sh.at[dst_v.at[c]], add=True)
            return 0

        lax.fori_loop(0, NCH, chunk, 0)
        plsc.subcore_barrier()
        pltpu.sync_copy(
            deg_sh.at[pl.ds(sid * RPT, RPT)],
            out_hbm.at[cid, pl.ds(sid * RPT, RPT)],
        )

    return k(dst)


def _scatter_call(src, dst, hn):
    src = src.reshape(NW, NCH // G, G, B)
    dst = dst.reshape(NW, NCH // G, G, B)

    @functools.partial(
        pl.kernel,
        out_type=jax.ShapeDtypeStruct((NC, NP, D), jnp.float32),
        mesh=_MESH,
        scratch_types=[
            pltpu.VMEM((2, G, B), jnp.int32),
            pltpu.VMEM((2, G, B), jnp.int32),
            pltpu.VMEM((B, D), jnp.float32),
            pltpu.VMEM((B, D), jnp.float32),
            pltpu.VMEM_SHARED((NP, D), jnp.float32),
            pltpu.SemaphoreType.DMA,
            pltpu.SemaphoreType.DMA,
            pltpu.SemaphoreType.DMA,
            pltpu.SemaphoreType.DMA,
            pltpu.SemaphoreType.DMA,
            pltpu.SemaphoreType.DMA,
        ],
    )
    def k(src_hbm, dst_hbm, hn_hbm, out_hbm, src_v, dst_v, rows_a, rows_b,
          acc_sh, sem_a, sem_b, ssem_a, ssem_b, isem_a, isem_b):
        cid = lax.axis_index("c")
        sid = lax.axis_index("s")
        wid = cid * NS + sid

        pltpu.async_copy(src_hbm.at[wid, 0], src_v.at[0], isem_a)
        pltpu.async_copy(dst_hbm.at[wid, 0], dst_v.at[0], isem_a)

        def zrow(r, _):
            for j in range(D // 16):
                rows_a[r, pl.ds(j * 16, 16)] = jnp.zeros((16,), jnp.float32)
            return 0

        lax.fori_loop(0, B, zrow, 0)
        for kblk in range(RPT // B):
            pltpu.async_copy(
                rows_a, acc_sh.at[pl.ds(sid * RPT + kblk * B, B)], ssem_a)
        if RPT % B:
            pltpu.async_copy(
                rows_a.at[pl.ds(0, RPT % B)],
                acc_sh.at[pl.ds(sid * RPT + (RPT // B) * B, RPT % B)],
                ssem_a)
        for kblk in range(RPT // B):
            pltpu.make_async_copy(
                rows_a, acc_sh.at[pl.ds(sid * RPT + kblk * B, B)],
                ssem_a).wait()
        if RPT % B:
            pltpu.make_async_copy(
                rows_a.at[pl.ds(0, RPT % B)],
                acc_sh.at[pl.ds(sid * RPT + (RPT // B) * B, RPT % B)],
                ssem_a).wait()
        plsc.subcore_barrier()

        def gather(p, c, buf, sem):
            return pltpu.async_copy(hn_hbm.at[src_v.at[p, c]], buf, sem)

        def gwait(buf, sem):
            pltpu.make_async_copy(hn_hbm.at[src_v.at[0, 0]], buf, sem).wait()

        def scat(p, c, buf, sem):
            return pltpu.async_copy(buf, acc_sh.at[dst_v.at[p, c]], sem,
                                    add=True)

        def swait(buf, sem):
            pltpu.make_async_copy(buf, acc_sh.at[dst_v.at[0, 0]], sem).wait()

        def iload(g, p, sem):
            pltpu.async_copy(src_hbm.at[wid, g], src_v.at[p], sem)
            pltpu.async_copy(dst_hbm.at[wid, g], dst_v.at[p], sem)

        def iwait(p, sem):
            pltpu.make_async_copy(src_hbm.at[wid, 0], src_v.at[p], sem).wait()
            pltpu.make_async_copy(dst_hbm.at[wid, 0], dst_v.at[p], sem).wait()

        def process(p):
            gather(p, 0, rows_a, sem_a)

            def pair(i, _):
                c = 2 * i
                gwait(rows_a, sem_a)
                scat(p, c, rows_a, ssem_a)
                gather(p, c + 1, rows_b, sem_b)
                swait(rows_a, ssem_a)
                gather(p, c + 2, rows_a, sem_a)
                gwait(rows_b, sem_b)
                scat(p, c + 1, rows_b, ssem_b)
                swait(rows_b, ssem_b)
                return 0

            lax.fori_loop(0, G // 2 - 1, pair, 0)
            gather(p, G - 1, rows_b, sem_b)
            gwait(rows_a, sem_a)
            pltpu.sync_copy(rows_a, acc_sh.at[dst_v.at[p, G - 2]], add=True)
            gwait(rows_b, sem_b)
            pltpu.sync_copy(rows_b, acc_sh.at[dst_v.at[p, G - 1]], add=True)

        def gpair(j, _):
            g = 2 * j
            iwait(0, isem_a)
            iload(g + 1, 1, isem_b)
            process(0)
            iwait(1, isem_b)
            iload(g + 2, 0, isem_a)
            process(1)
            return 0

        lax.fori_loop(0, (NCH // G) // 2, gpair, 0)
        iwait(0, isem_a)
        process(0)
        plsc.subcore_barrier()
        pltpu.sync_copy(
            acc_sh.at[pl.ds(sid * RPT, RPT)],
            out_hbm.at[cid, pl.ds(sid * RPT, RPT)],
        )

    return k(src, dst, hn)


def _dinv_of(deg_ref):
    return lax.rsqrt(deg_ref[0] + deg_ref[1] + 1.0)


def _tc_first_body(deg_ref, x_ref, w_ref, o_ref):
    dinv = _dinv_of(deg_ref)
    h = jnp.dot(x_ref[...], w_ref[...], preferred_element_type=jnp.float32)
    o_ref[...] = h * dinv


def _tc_mid_body(deg_ref, acc_ref, hn_ref, w_ref, s_ref, sh_ref, o_ref):
    dinv = _dinv_of(deg_ref)
    tot = (acc_ref[0] + acc_ref[1] + hn_ref[...]) * dinv
    f = jnp.maximum(tot * s_ref[...] + sh_ref[...], 0.0)
    h = jnp.dot(f, w_ref[...], preferred_element_type=jnp.float32)
    o_ref[...] = h * dinv


def _tc_last_body(deg_ref, acc_ref, hn_ref, b_ref, o_ref):
    dinv = _dinv_of(deg_ref)
    o_ref[...] = (acc_ref[0] + acc_ref[1] + hn_ref[...]) * dinv + b_ref[...]


_deg_spec = pl.BlockSpec((2, RB, 1), lambda i: (0, i, 0))
_acc_spec = pl.BlockSpec((2, RB, D), lambda i: (0, i, 0))
_row_spec = pl.BlockSpec((RB, D), lambda i: (i, 0))
_w_spec = pl.BlockSpec((D, D), lambda i: (0, 0))
_v_spec = pl.BlockSpec((1, D), lambda i: (0, 0))
_OUT = jax.ShapeDtypeStruct((N, D), jnp.float32)


def _tc_first(deg3, x, w):
    return pl.pallas_call(
        _tc_first_body,
        grid=(N // RB,),
        in_specs=[_deg_spec, _row_spec, _w_spec],
        out_specs=_row_spec,
        out_shape=_OUT,
    )(deg3, x, w)


def _tc_mid(deg3, acc, hn, w, s, sh):
    return pl.pallas_call(
        _tc_mid_body,
        grid=(N // RB,),
        in_specs=[_deg_spec, _acc_spec, _row_spec, _w_spec, _v_spec, _v_spec],
        out_specs=_row_spec,
        out_shape=_OUT,
    )(deg3, acc, hn, w, s, sh)


def _tc_last(deg3, acc, hn, b):
    return pl.pallas_call(
        _tc_last_body,
        grid=(N // RB,),
        in_specs=[_deg_spec, _acc_spec, _row_spec, _v_spec],
        out_specs=_row_spec,
        out_shape=_OUT,
    )(deg3, acc, hn, b)


def kernel(x, edge_index, W1, b1, g1, be1, W2, b2, g2, be2, Wo, bo):
    src = edge_index[0].reshape(NW, NCH, B)
    dst = edge_index[1].reshape(NW, NCH, B)

    s_bn = 1.0 / jnp.sqrt(jnp.float32(1.0 + EPS))
    s1 = (g1 * s_bn).reshape(1, D)
    sh1 = (b1 * g1 * s_bn + be1).reshape(1, D)
    s2 = (g2 * s_bn).reshape(1, D)
    sh2 = (b2 * g2 * s_bn + be2).reshape(1, D)

    degs = _deg_call(dst)
    deg3 = degs[:, :, :1]

    hn1 = _tc_first(deg3, x, W1)
    acc1 = _scatter_call(src, dst, hn1)
    hn2 = _tc_mid(deg3, acc1, hn1, W2, s1, sh1)
    acc2 = _scatter_call(src, dst, hn2)
    hn3 = _tc_mid(deg3, acc2, hn2, Wo, s2, sh2)
    acc3 = _scatter_call(src, dst, hn3)
    return _tc_last(deg3, acc3, hn3, bo.reshape(1, D))


# speedup vs baseline: 1.0031x; 1.0031x over previous
import functools

import jax
import jax.numpy as jnp
from jax import lax
from jax.experimental import pallas as pl
from jax.experimental.pallas import tpu as pltpu
from jax.experimental.pallas import tpu_sc as plsc

N = 10000
E = 320000
D = 128
EPS = 1e-5

NC = 2
NS = 16
NW = NC * NS
EPT = E // NW
B = 125
NCH = EPT // B
G = 16
NP = 10240
RPT = NP // NS
RB = 10000

_MESH = plsc.VectorSubcoreMesh(
    core_axis_name="c", subcore_axis_name="s", num_cores=NC, num_subcores=NS
)


def _deg_call(dst):
    @functools.partial(
        pl.kernel,
        out_type=jax.ShapeDtypeStruct((NC, NP, D), jnp.float32),
        mesh=_MESH,
        scratch_types=[
            pltpu.VMEM((NCH, B), jnp.int32),
            pltpu.VMEM((B, D), jnp.float32),
            pltpu.VMEM_SHARED((NP, D), jnp.float32),
            pltpu.SemaphoreType.DMA,
        ],
    )
    def k(dst_hbm, out_hbm, dst_v, buf_v, deg_sh, isem):
        cid = lax.axis_index("c")
        sid = lax.axis_index("s")
        wid = cid * NS + sid

        pltpu.async_copy(dst_hbm.at[wid], dst_v, isem)

        def fillz(r, _):
            for j in range(D // 16):
                buf_v[r, pl.ds(j * 16, 16)] = jnp.zeros((16,), jnp.float32)
            return 0

        lax.fori_loop(0, B, fillz, 0)
        for kblk in range(RPT // B):
            pltpu.sync_copy(buf_v, deg_sh.at[pl.ds(sid * RPT + kblk * B, B)])
        if RPT % B:
            pltpu.sync_copy(
                buf_v.at[pl.ds(0, RPT % B)],
                deg_sh.at[pl.ds(sid * RPT + (RPT // B) * B, RPT % B)],
            )

        def fillo(r, _):
            for j in range(D // 16):
                buf_v[r, pl.ds(j * 16, 16)] = jnp.ones((16,), jnp.float32)
            return 0

        lax.fori_loop(0, B, fillo, 0)
        pltpu.make_async_copy(dst_hbm.at[wid], dst_v, isem).wait()
        plsc.subcore_barrier()

        def chunk(c, _):
            pltpu.async_copy(buf_v, deg_sh.at[dst_v.at[c]], isem, add=True)
            return 0

        lax.fori_loop(0, NCH, chunk, 0)

        def drain(c, _):
            pltpu.make_async_copy(buf_v, deg_sh.at[dst_v.at[0]], isem).wait()
            return 0

        lax.fori_loop(0, NCH, drain, 0)
        plsc.subcore_barrier()
        pltpu.sync_copy(
            deg_sh.at[pl.ds(sid * RPT, RPT)],
            out_hbm.at[cid, pl.ds(sid * RPT, RPT)],
        )

    return k(dst)


def _scatter_call(src, dst, hn):
    src = src.reshape(NW, NCH // G, G, B)
    dst = dst.reshape(NW, NCH // G, G, B)

    @functools.partial(
        pl.kernel,
        out_type=jax.ShapeDtypeStruct((NC, NP, D), jnp.float32),
        mesh=_MESH,
        scratch_types=[
            pltpu.VMEM((2, G, B), jnp.int32),
            pltpu.VMEM((2, G, B), jnp.int32),
            pltpu.VMEM((B, D), jnp.float32),
            pltpu.VMEM((B, D), jnp.float32),
            pltpu.VMEM_SHARED((NP, D), jnp.float32),
            pltpu.SemaphoreType.DMA,
            pltpu.SemaphoreType.DMA,
            pltpu.SemaphoreType.DMA,
            pltpu.SemaphoreType.DMA,
            pltpu.SemaphoreType.DMA,
            pltpu.SemaphoreType.DMA,
        ],
    )
    def k(src_hbm, dst_hbm, hn_hbm, out_hbm, src_v, dst_v, rows_a, rows_b,
          acc_sh, sem_a, sem_b, ssem_a, ssem_b, isem_a, isem_b):
        cid = lax.axis_index("c")
        sid = lax.axis_index("s")
        wid = cid * NS + sid

        pltpu.async_copy(src_hbm.at[wid, 0], src_v.at[0], isem_a)
        pltpu.async_copy(dst_hbm.at[wid, 0], dst_v.at[0], isem_a)

        def zrow(r, _):
            for j in range(D // 16):
                rows_a[r, pl.ds(j * 16, 16)] = jnp.zeros((16,), jnp.float32)
            return 0

        lax.fori_loop(0, B, zrow, 0)
        for kblk in range(RPT // B):
            pltpu.async_copy(
                rows_a, acc_sh.at[pl.ds(sid * RPT + kblk * B, B)], ssem_a)
        if RPT % B:
            pltpu.async_copy(
                rows_a.at[pl.ds(0, RPT % B)],
                acc_sh.at[pl.ds(sid * RPT + (RPT // B) * B, RPT % B)],
                ssem_a)
        for kblk in range(RPT // B):
            pltpu.make_async_copy(
                rows_a, acc_sh.at[pl.ds(sid * RPT + kblk * B, B)],
                ssem_a).wait()
        if RPT % B:
            pltpu.make_async_copy(
                rows_a.at[pl.ds(0, RPT % B)],
                acc_sh.at[pl.ds(sid * RPT + (RPT // B) * B, RPT % B)],
                ssem_a).wait()
        plsc.subcore_barrier()

        def gather(p, c, buf, sem):
            return pltpu.async_copy(hn_hbm.at[src_v.at[p, c]], buf, sem)

        def gwait(buf, sem):
            pltpu.make_async_copy(hn_hbm.at[src_v.at[0, 0]], buf, sem).wait()

        def scat(p, c, buf, sem):
            return pltpu.async_copy(buf, acc_sh.at[dst_v.at[p, c]], sem,
                                    add=True)

        def swait(buf, sem):
            pltpu.make_async_copy(buf, acc_sh.at[dst_v.at[0, 0]], sem).wait()

        def iload(g, p, sem):
            pltpu.async_copy(src_hbm.at[wid, g], src_v.at[p], sem)
            pltpu.async_copy(dst_hbm.at[wid, g], dst_v.at[p], sem)

        def iwait(p, sem):
            pltpu.make_async_copy(src_hbm.at[wid, 0], src_v.at[p], sem).wait()
            pltpu.make_async_copy(dst_hbm.at[wid, 0], dst_v.at[p], sem).wait()

        def process(p):
            gather(p, 0, rows_a, sem_a)

            def pair(i, _):
                c = 2 * i
                gwait(rows_a, sem_a)
                scat(p, c, rows_a, ssem_a)
                gather(p, c + 1, rows_b, sem_b)
                swait(rows_a, ssem_a)
                gather(p, c + 2, rows_a, sem_a)
                gwait(rows_b, sem_b)
                scat(p, c + 1, rows_b, ssem_b)
                swait(rows_b, ssem_b)
                return 0

            lax.fori_loop(0, G // 2 - 1, pair, 0)
            gather(p, G - 1, rows_b, sem_b)
            gwait(rows_a, sem_a)
            pltpu.sync_copy(rows_a, acc_sh.at[dst_v.at[p, G - 2]], add=True)
            gwait(rows_b, sem_b)
            pltpu.sync_copy(rows_b, acc_sh.at[dst_v.at[p, G - 1]], add=True)

        def gpair(j, _):
            g = 2 * j
            iwait(0, isem_a)
            iload(g + 1, 1, isem_b)
            process(0)
            iwait(1, isem_b)
            iload(g + 2, 0, isem_a)
            process(1)
            return 0

        lax.fori_loop(0, (NCH // G) // 2, gpair, 0)
        iwait(0, isem_a)
        process(0)
        plsc.subcore_barrier()
        pltpu.sync_copy(
            acc_sh.at[pl.ds(sid * RPT, RPT)],
            out_hbm.at[cid, pl.ds(sid * RPT, RPT)],
        )

    return k(src, dst, hn)


def _dinv_of(deg_ref):
    return lax.rsqrt(deg_ref[0] + deg_ref[1] + 1.0)


def _tc_first_body(deg_ref, x_ref, w_ref, o_ref):
    dinv = _dinv_of(deg_ref)
    h = jnp.dot(x_ref[...], w_ref[...], preferred_element_type=jnp.float32)
    o_ref[...] = h * dinv


def _tc_mid_body(deg_ref, acc_ref, hn_ref, w_ref, s_ref, sh_ref, o_ref):
    dinv = _dinv_of(deg_ref)
    tot = (acc_ref[0] + acc_ref[1] + hn_ref[...]) * dinv
    f = jnp.maximum(tot * s_ref[...] + sh_ref[...], 0.0)
    h = jnp.dot(f, w_ref[...], preferred_element_type=jnp.float32)
    o_ref[...] = h * dinv


def _tc_last_body(deg_ref, acc_ref, hn_ref, b_ref, o_ref):
    dinv = _dinv_of(deg_ref)
    o_ref[...] = (acc_ref[0] + acc_ref[1] + hn_ref[...]) * dinv + b_ref[...]


_deg_spec = pl.BlockSpec((2, RB, 1), lambda i: (0, i, 0))
_acc_spec = pl.BlockSpec((2, RB, D), lambda i: (0, i, 0))
_row_spec = pl.BlockSpec((RB, D), lambda i: (i, 0))
_w_spec = pl.BlockSpec((D, D), lambda i: (0, 0))
_v_spec = pl.BlockSpec((1, D), lambda i: (0, 0))
_OUT = jax.ShapeDtypeStruct((N, D), jnp.float32)


def _tc_first(deg3, x, w):
    return pl.pallas_call(
        _tc_first_body,
        grid=(N // RB,),
        in_specs=[_deg_spec, _row_spec, _w_spec],
        out_specs=_row_spec,
        out_shape=_OUT,
    )(deg3, x, w)


def _tc_mid(deg3, acc, hn, w, s, sh):
    return pl.pallas_call(
        _tc_mid_body,
        grid=(N // RB,),
        in_specs=[_deg_spec, _acc_spec, _row_spec, _w_spec, _v_spec, _v_spec],
        out_specs=_row_spec,
        out_shape=_OUT,
    )(deg3, acc, hn, w, s, sh)


def _tc_last(deg3, acc, hn, b):
    return pl.pallas_call(
        _tc_last_body,
        grid=(N // RB,),
        in_specs=[_deg_spec, _acc_spec, _row_spec, _v_spec],
        out_specs=_row_spec,
        out_shape=_OUT,
    )(deg3, acc, hn, b)


def kernel(x, edge_index, W1, b1, g1, be1, W2, b2, g2, be2, Wo, bo):
    src = edge_index[0].reshape(NW, NCH, B)
    dst = edge_index[1].reshape(NW, NCH, B)

    s_bn = 1.0 / jnp.sqrt(jnp.float32(1.0 + EPS))
    s1 = (g1 * s_bn).reshape(1, D)
    sh1 = (b1 * g1 * s_bn + be1).reshape(1, D)
    s2 = (g2 * s_bn).reshape(1, D)
    sh2 = (b2 * g2 * s_bn + be2).reshape(1, D)

    degs = _deg_call(dst)
    deg3 = degs[:, :, :1]

    hn1 = _tc_first(deg3, x, W1)
    acc1 = _scatter_call(src, dst, hn1)
    hn2 = _tc_mid(deg3, acc1, hn1, W2, s1, sh1)
    acc2 = _scatter_call(src, dst, hn2)
    hn3 = _tc_mid(deg3, acc2, hn2, Wo, s2, sh2)
    acc3 = _scatter_call(src, dst, hn3)
    return _tc_last(deg3, acc3, hn3, bo.reshape(1, D))
